# baseline (speedup 1.0000x reference)
import functools

import jax
import jax.numpy as jnp
from jax import lax
from jax.experimental import pallas as pl
from jax.experimental.pallas import tpu as pltpu
from jax.experimental.pallas import tpu_sc as plsc

_LANES = 16
_IDXW = 128
_E = 320000
_N = 10000
_IDX_ROWS = _E // _IDXW
N_PAD = 16 * 626
_P1_BASE = _IDX_ROWS // 16
_P1_REM = _IDX_ROWS - 16 * _P1_BASE
_P2_BASE = _IDX_ROWS // 32
_P2_REM = _IDX_ROWS - 32 * _P2_BASE
_P1_CHUNK = 26
_P2_CHUNK = 26
_G = _P1_CHUNK * _IDXW // 8


@functools.partial(
    pl.kernel,
    out_type=jax.ShapeDtypeStruct((_E * _LANES // 128, 128), jnp.float32),
    mesh=plsc.VectorSubcoreMesh(
        core_axis_name="c", subcore_axis_name="s", num_cores=2, num_subcores=16
    ),
    scratch_types=[
        pltpu.VMEM_SHARED((N_PAD, _LANES), jnp.float32),
        pltpu.VMEM((_G, 128), jnp.float32),
        pltpu.VMEM((_P1_CHUNK * _IDXW, _LANES), jnp.float32),
        pltpu.VMEM((_P1_CHUNK, _IDXW), jnp.int32),
    ],
    compiler_params=pltpu.CompilerParams(use_tc_tiling_on_sc=False),
)
def _sc_dmpnn(efeat128_hbm, eidx_hbm, out128_hbm, neigh, wbuf, tbuf, idx_v):
    c = lax.axis_index("c")
    s = lax.axis_index("s")
    dst_hbm = eidx_hbm.at[1]
    src_hbm = eidx_hbm.at[0]

    zrows = N_PAD // 16

    def _zero(i, carry):
        tbuf[i] = jnp.zeros((_LANES,), jnp.float32)
        return carry

    lax.fori_loop(0, zrows, _zero, 0)
    pltpu.sync_copy(tbuf.at[pl.ds(0, zrows)], neigh.at[pl.ds(s * zrows, zrows)])
    plsc.subcore_barrier()

    def _p1_chunk(rbase, n_idx_rows):
        n_e = n_idx_rows * _IDXW
        pltpu.sync_copy(
            dst_hbm.at[pl.ds(rbase, n_idx_rows)], idx_v.at[pl.ds(0, n_idx_rows)]
        )
        pltpu.sync_copy(
            efeat128_hbm.at[pl.ds(rbase * 16, n_idx_rows * 16)],
            wbuf.at[pl.ds(0, n_idx_rows * 16)],
        )

        def _repack(g, carry):
            for q in range(8):
                tbuf[8 * g + q] = wbuf[g, 16 * q : 16 * (q + 1)]
            return carry

        lax.fori_loop(0, n_e // 8, _repack, 0)
        for j in range(n_idx_rows):
            pltpu.sync_copy(
                tbuf.at[pl.ds(j * _IDXW, _IDXW)], neigh.at[idx_v.at[j]], add=True
            )

    for chunk in range(_P1_BASE // _P1_CHUNK):
        _p1_chunk(s * _P1_BASE + chunk * _P1_CHUNK, _P1_CHUNK)

    @pl.when(s < _P1_REM)
    def _p1_rem():
        _p1_chunk(16 * _P1_BASE + s, 1)

    plsc.subcore_barrier()

    wid = c * 16 + s

    def _p2_chunk(rbase, n_idx_rows):
        n_e = n_idx_rows * _IDXW
        pltpu.sync_copy(
            src_hbm.at[pl.ds(rbase, n_idx_rows)], idx_v.at[pl.ds(0, n_idx_rows)]
        )
        for j in range(n_idx_rows):
            pltpu.sync_copy(
                neigh.at[idx_v.at[j]], tbuf.at[pl.ds(j * _IDXW, _IDXW)]
            )
        pltpu.sync_copy(
            efeat128_hbm.at[pl.ds(rbase * 16, n_idx_rows * 16)],
            wbuf.at[pl.ds(0, n_idx_rows * 16)],
        )

        def _sub(g, carry):
            for q in range(4):
                w0 = wbuf[g, 32 * q : 32 * q + 16]
                w1 = wbuf[g, 32 * q + 16 : 32 * q + 32]
                t0 = tbuf[8 * g + 2 * q]
                t1 = tbuf[8 * g + 2 * q + 1]
                wbuf[g, 32 * q : 32 * q + 16] = t0 - w1
                wbuf[g, 32 * q + 16 : 32 * q + 32] = t1 - w0
            return carry

        lax.fori_loop(0, n_e // 8, _sub, 0)
        pltpu.sync_copy(
            wbuf.at[pl.ds(0, n_idx_rows * 16)],
            out128_hbm.at[pl.ds(rbase * 16, n_idx_rows * 16)],
        )

    for chunk in range(_P2_BASE // _P2_CHUNK):
        _p2_chunk(wid * _P2_BASE + chunk * _P2_CHUNK, _P2_CHUNK)

    @pl.when(wid < _P2_REM)
    def _p2_rem():
        _p2_chunk(32 * _P2_BASE + wid, 1)


def kernel(nfeat, efeat, edge_index):
    eidx = edge_index.reshape(2, _IDX_ROWS, _IDXW)
    efeat128 = efeat.reshape(_E * _LANES // 128, 128)
    out128 = _sc_dmpnn(efeat128, eidx)
    return out128.reshape(_E, _LANES)
